# baseline (speedup 1.0000x reference)
import functools

import jax
import jax.numpy as jnp
from jax import lax
from jax.experimental import pallas as pl
from jax.experimental.pallas import tpu as pltpu
from jax.experimental.pallas import tpu_sc as plsc

N = 10000
E = 320000
D_IN = 128
K = 32
KW = 128
NC = 2
NS = 16
NW = NC * NS
CHUNK = 128
HSTR = 64
CPW = 80
EPW = CHUNK * CPW
EPAD = EPW * NW
ROWS_A = 624
ROWS_B = N - (NS - 1) * ROWS_A
NBUF = 2
PIECE = 16

_sc_mesh = plsc.VectorSubcoreMesh(core_axis_name="c", subcore_axis_name="s")


@functools.partial(
    pl.kernel,
    out_type=jax.ShapeDtypeStruct((NC, N, KW), jnp.float32),
    mesh=_sc_mesh,
    scratch_types=[
        pltpu.VMEM((2 * PIECE, HSTR), jnp.int32),
        pltpu.VMEM((2 * PIECE, HSTR), jnp.int32),
        pltpu.VMEM((PIECE, CHUNK), jnp.float32),
        [pltpu.VMEM((CHUNK, KW), jnp.float32) for _ in range(NBUF)],
        pltpu.VMEM((16, KW), jnp.float32),
        pltpu.VMEM_SHARED((N, KW), jnp.float32),
        [[pltpu.SemaphoreType.DMA for _ in range(2)] for _ in range(NBUF)],
        [[pltpu.SemaphoreType.DMA for _ in range(2)] for _ in range(NBUF)],
    ],
)
def _sc_spmm(sup_hbm, src_hbm, dst_hbm, w_hbm, out_hbm,
             src_v, dst_v, w_v, rows, zblk_v, accum, gsem, ssem):
    c = lax.axis_index("c")
    s = lax.axis_index("s")
    wid = c * NS + s

    off = pl.multiple_of(s * ROWS_A, 8)

    zv = jnp.zeros((16,), jnp.float32)
    for r in range(16):
        for q in range(KW // 16):
            zblk_v[r, pl.ds(q * 16, 16)] = zv

    def zero_body(i, carry):
        pltpu.sync_copy(
            zblk_v, accum.at[pl.ds(pl.multiple_of(off + i * 16, 8), 16)])
        return carry

    nzero = jnp.where(s < NS - 1, ROWS_A // 16, ROWS_B // 16)
    lax.fori_loop(0, nzero, zero_body, 0)
    plsc.subcore_barrier()

    def _gather(j, b):
        for u in range(2):
            pltpu.async_copy(sup_hbm.at[src_v.at[2 * j + u]],
                             rows[b].at[pl.ds(u * HSTR, HSTR)], gsem[b][u])

    def _gather_wait(j, b):
        for u in range(2):
            pltpu.make_async_copy(sup_hbm.at[src_v.at[2 * j + u]],
                                  rows[b].at[pl.ds(u * HSTR, HSTR)],
                                  gsem[b][u]).wait()

    def _scatter(j, b):
        for u in range(2):
            pltpu.async_copy(rows[b].at[pl.ds(u * HSTR, HSTR)],
                             accum.at[dst_v.at[2 * j + u]], ssem[b][u],
                             add=True)

    def _scatter_wait(j, b):
        for u in range(2):
            pltpu.make_async_copy(rows[b].at[pl.ds(u * HSTR, HSTR)],
                                  accum.at[dst_v.at[2 * j + u]],
                                  ssem[b][u]).wait()

    def _scale(j, buf):
        def scale_body(g, carry2):
            wv = w_v[j, pl.ds(g * 16, 16)]
            base = g * 16
            for l in range(16):
                w = wv[l]
                buf[base + l, pl.ds(0, 16)] = buf[base + l, pl.ds(0, 16)] * w
                buf[base + l, pl.ds(16, 16)] = buf[base + l, pl.ds(16, 16)] * w
            return carry2

        lax.fori_loop(0, CHUNK // 16, scale_body, 0)

    for h in range(CPW // PIECE):
        pltpu.sync_copy(src_hbm.at[wid, pl.ds(h * 2 * PIECE, 2 * PIECE)],
                        src_v)
        pltpu.sync_copy(dst_hbm.at[wid, pl.ds(h * 2 * PIECE, 2 * PIECE)],
                        dst_v)
        pltpu.sync_copy(w_hbm.at[wid, pl.ds(h * PIECE, PIECE)], w_v)

        for b in range(NBUF):
            _gather(b, b)

        def chunk_body(jj, carry):
            for b in range(NBUF):
                j = jj * NBUF + b
                _gather_wait(j, b)
                _scale(j, rows[b])
                _scatter(j, b)

            @pl.when(jj < PIECE // NBUF - 1)
            def _():
                for b in range(NBUF):
                    j = jj * NBUF + b
                    _scatter_wait(j, b)
                    _gather(j + NBUF, b)

            return carry

        lax.fori_loop(0, PIECE // NBUF, chunk_body, 0)
        for b in range(NBUF):
            _scatter_wait(PIECE - NBUF + b, b)
    plsc.subcore_barrier()

    @pl.when(s < NS - 1)
    def _():
        pltpu.sync_copy(accum.at[pl.ds(off, ROWS_A)],
                        out_hbm.at[c, pl.ds(off, ROWS_A)])

    @pl.when(s == NS - 1)
    def _():
        pltpu.sync_copy(accum.at[pl.ds(off, ROWS_B)],
                        out_hbm.at[c, pl.ds(off, ROWS_B)])


def _mm1_body(x_ref, w_ref, o_ref):
    h = jnp.dot(x_ref[...], w_ref[...], preferred_element_type=jnp.float32)
    o_ref[...] = jnp.concatenate(
        [h, jnp.zeros((h.shape[0], KW - K), jnp.float32)], axis=1)


def _tc_mm1(x, w1):
    bm = 2000
    return pl.pallas_call(
        _mm1_body,
        grid=(N // bm,),
        in_specs=[pl.BlockSpec((bm, D_IN), lambda i: (i, 0)),
                  pl.BlockSpec((D_IN, K), lambda i: (0, 0))],
        out_specs=pl.BlockSpec((bm, KW), lambda i: (i, 0)),
        out_shape=jax.ShapeDtypeStruct((N, KW), jnp.float32),
    )(x, w1)


def _cmb_body(p0_ref, p1_ref, w_ref, o_ref):
    h = jnp.maximum(p0_ref[:, :K] + p1_ref[:, :K], 0.0)
    h = jnp.dot(h, w_ref[...], preferred_element_type=jnp.float32)
    o_ref[...] = jnp.concatenate(
        [h, jnp.zeros((h.shape[0], KW - K), jnp.float32)], axis=1)


def _tc_relu_mm(p0, p1, w23):
    bm = 2000
    return pl.pallas_call(
        _cmb_body,
        grid=(N // bm,),
        in_specs=[pl.BlockSpec((bm, KW), lambda i: (i, 0)),
                  pl.BlockSpec((bm, KW), lambda i: (i, 0)),
                  pl.BlockSpec((K, K), lambda i: (0, 0))],
        out_specs=pl.BlockSpec((bm, KW), lambda i: (i, 0)),
        out_shape=jax.ShapeDtypeStruct((N, KW), jnp.float32),
    )(p0, p1, w23)


def _add_body(a_ref, b_ref, o_ref):
    o_ref[...] = a_ref[:, :K] + b_ref[:, :K]


def _tc_add(a, b):
    bm = 2000
    return pl.pallas_call(
        _add_body,
        grid=(N // bm,),
        in_specs=[pl.BlockSpec((bm, KW), lambda i: (i, 0)),
                  pl.BlockSpec((bm, KW), lambda i: (i, 0))],
        out_specs=pl.BlockSpec((bm, K), lambda i: (i, 0)),
        out_shape=jax.ShapeDtypeStruct((N, K), jnp.float32),
    )(a, b)


def _dec_body(a_ref, bt_ref, o_ref):
    o_ref[...] = jax.nn.sigmoid(
        jnp.dot(a_ref[...], bt_ref[...], preferred_element_type=jnp.float32))


def _tc_decoder(mu, mu_t):
    bm = 1024
    bn = 1024
    return pl.pallas_call(
        _dec_body,
        grid=(pl.cdiv(N, bm), pl.cdiv(N, bn)),
        in_specs=[pl.BlockSpec((bm, 16), lambda i, j: (i, 0)),
                  pl.BlockSpec((16, bn), lambda i, j: (0, j))],
        out_specs=pl.BlockSpec((bm, bn), lambda i, j: (i, j)),
        out_shape=jax.ShapeDtypeStruct((N, N), jnp.float32),
    )(mu, mu_t)


def kernel(x, edge_index, edge_weight, W1, W2, W3):
    src = edge_index[0]
    dst = edge_index[1]
    pad = EPAD - E
    src3 = jnp.pad(src, (0, pad)).reshape(NW, 2 * CPW, HSTR)
    dst3 = jnp.pad(dst, (0, pad)).reshape(NW, 2 * CPW, HSTR)
    w3 = jnp.pad(edge_weight, (0, pad)).reshape(NW, CPW, CHUNK)

    sup1 = _tc_mm1(x, W1)
    p = _sc_spmm(sup1, src3, dst3, w3)
    w23 = jnp.concatenate([W2, W3], axis=1)
    sup23 = _tc_relu_mm(p[0], p[1], w23)
    q = _sc_spmm(sup23, src3, dst3, w3)
    z = _tc_add(q[0], q[1])
    mu = z[:, :16]
    logvar = z[:, 16:]
    adj = _tc_decoder(mu, mu.T)
    return adj, mu, logvar


# speedup vs baseline: 1.1968x; 1.1968x over previous
import functools

import jax
import jax.numpy as jnp
from jax import lax
from jax.experimental import pallas as pl
from jax.experimental.pallas import tpu as pltpu
from jax.experimental.pallas import tpu_sc as plsc

N = 10000
E = 320000
D_IN = 128
K = 32
KW = 128
NC = 2
NS = 16
NW = NC * NS
CHUNK = 128
CPW = 80
EPW = CHUNK * CPW
EPAD = EPW * NW
ROWS_A = 624
ROWS_B = N - (NS - 1) * ROWS_A
NBUF = 2
HALF = CPW // 2

_sc_mesh = plsc.VectorSubcoreMesh(core_axis_name="c", subcore_axis_name="s")


@functools.partial(
    pl.kernel,
    out_type=jax.ShapeDtypeStruct((NC, N, KW), jnp.float32),
    mesh=_sc_mesh,
    scratch_types=[
        pltpu.VMEM((HALF, CHUNK), jnp.int32),
        pltpu.VMEM((HALF, CHUNK), jnp.int32),
        pltpu.VMEM((HALF, CHUNK), jnp.float32),
        [pltpu.VMEM((CHUNK, KW), jnp.float32) for _ in range(NBUF)],
        pltpu.VMEM((16, KW), jnp.float32),
        pltpu.VMEM_SHARED((N, KW), jnp.float32),
        [pltpu.SemaphoreType.DMA for _ in range(NBUF)],
        [pltpu.SemaphoreType.DMA for _ in range(NBUF)],
    ],
)
def _sc_spmm(sup_hbm, src_hbm, dst_hbm, w_hbm, out_hbm,
             src_v, dst_v, w_v, rows, zblk_v, accum, gsem, ssem):
    c = lax.axis_index("c")
    s = lax.axis_index("s")
    wid = c * NS + s

    off = pl.multiple_of(s * ROWS_A, 8)

    zv = jnp.zeros((16,), jnp.float32)
    for r in range(16):
        for q in range(KW // 16):
            zblk_v[r, pl.ds(q * 16, 16)] = zv

    def zero_body(i, carry):
        pltpu.sync_copy(
            zblk_v, accum.at[pl.ds(pl.multiple_of(off + i * 16, 8), 16)])
        return carry

    nzero = jnp.where(s < NS - 1, ROWS_A // 16, ROWS_B // 16)
    lax.fori_loop(0, nzero, zero_body, 0)
    plsc.subcore_barrier()

    def _gather(j, b):
        pltpu.async_copy(sup_hbm.at[src_v.at[j]], rows[b], gsem[b])

    def _gather_wait(j, b):
        pltpu.make_async_copy(sup_hbm.at[src_v.at[j]], rows[b],
                              gsem[b]).wait()

    def _scatter(j, b):
        pltpu.async_copy(rows[b], accum.at[dst_v.at[j]], ssem[b], add=True)

    def _scatter_wait(j, b):
        pltpu.make_async_copy(rows[b], accum.at[dst_v.at[j]],
                              ssem[b]).wait()

    def _scale(j, buf):
        def scale_body(g, carry2):
            wv = w_v[j, pl.ds(g * 16, 16)]
            base = g * 16
            for l in range(16):
                w = wv[l]
                buf[base + l, pl.ds(0, 16)] = buf[base + l, pl.ds(0, 16)] * w
                buf[base + l, pl.ds(16, 16)] = buf[base + l, pl.ds(16, 16)] * w
            return carry2

        lax.fori_loop(0, CHUNK // 16, scale_body, 0)

    for h in range(2):
        pltpu.sync_copy(src_hbm.at[wid, pl.ds(h * HALF, HALF)], src_v)
        pltpu.sync_copy(dst_hbm.at[wid, pl.ds(h * HALF, HALF)], dst_v)
        pltpu.sync_copy(w_hbm.at[wid, pl.ds(h * HALF, HALF)], w_v)

        for b in range(NBUF):
            _gather(b, b)

        def chunk_body(jj, carry):
            for b in range(NBUF):
                j = jj * NBUF + b
                _gather_wait(j, b)
                _scale(j, rows[b])
                _scatter(j, b)

            @pl.when(jj < HALF // NBUF - 1)
            def _():
                for b in range(NBUF):
                    j = jj * NBUF + b
                    _scatter_wait(j, b)
                    _gather(j + NBUF, b)

            return carry

        lax.fori_loop(0, HALF // NBUF, chunk_body, 0)
        for b in range(NBUF):
            _scatter_wait(HALF - NBUF + b, b)
    plsc.subcore_barrier()

    @pl.when(s < NS - 1)
    def _():
        pltpu.sync_copy(accum.at[pl.ds(off, ROWS_A)],
                        out_hbm.at[c, pl.ds(off, ROWS_A)])

    @pl.when(s == NS - 1)
    def _():
        pltpu.sync_copy(accum.at[pl.ds(off, ROWS_B)],
                        out_hbm.at[c, pl.ds(off, ROWS_B)])


def _mm1_body(x_ref, w_ref, o_ref):
    h = jnp.dot(x_ref[...], w_ref[...], preferred_element_type=jnp.float32)
    o_ref[...] = jnp.concatenate(
        [h, jnp.zeros((h.shape[0], KW - K), jnp.float32)], axis=1)


def _tc_mm1(x, w1):
    bm = 2000
    return pl.pallas_call(
        _mm1_body,
        grid=(N // bm,),
        in_specs=[pl.BlockSpec((bm, D_IN), lambda i: (i, 0)),
                  pl.BlockSpec((D_IN, K), lambda i: (0, 0))],
        out_specs=pl.BlockSpec((bm, KW), lambda i: (i, 0)),
        out_shape=jax.ShapeDtypeStruct((N, KW), jnp.float32),
    )(x, w1)


def _cmb_body(p0_ref, p1_ref, w_ref, o_ref):
    h = jnp.maximum(p0_ref[:, :K] + p1_ref[:, :K], 0.0)
    h = jnp.dot(h, w_ref[...], preferred_element_type=jnp.float32)
    o_ref[...] = jnp.concatenate(
        [h, jnp.zeros((h.shape[0], KW - K), jnp.float32)], axis=1)


def _tc_relu_mm(p0, p1, w23):
    bm = 2000
    return pl.pallas_call(
        _cmb_body,
        grid=(N // bm,),
        in_specs=[pl.BlockSpec((bm, KW), lambda i: (i, 0)),
                  pl.BlockSpec((bm, KW), lambda i: (i, 0)),
                  pl.BlockSpec((K, K), lambda i: (0, 0))],
        out_specs=pl.BlockSpec((bm, KW), lambda i: (i, 0)),
        out_shape=jax.ShapeDtypeStruct((N, KW), jnp.float32),
    )(p0, p1, w23)


def _add_body(a_ref, b_ref, o_ref):
    o_ref[...] = a_ref[:, :K] + b_ref[:, :K]


def _tc_add(a, b):
    bm = 2000
    return pl.pallas_call(
        _add_body,
        grid=(N // bm,),
        in_specs=[pl.BlockSpec((bm, KW), lambda i: (i, 0)),
                  pl.BlockSpec((bm, KW), lambda i: (i, 0))],
        out_specs=pl.BlockSpec((bm, K), lambda i: (i, 0)),
        out_shape=jax.ShapeDtypeStruct((N, K), jnp.float32),
    )(a, b)


def _dec_body(a_ref, bt_ref, o_ref):
    o_ref[...] = jax.nn.sigmoid(
        jnp.dot(a_ref[...], bt_ref[...], preferred_element_type=jnp.float32))


def _tc_decoder(mu, mu_t):
    bm = 1024
    bn = 1024
    return pl.pallas_call(
        _dec_body,
        grid=(pl.cdiv(N, bm), pl.cdiv(N, bn)),
        in_specs=[pl.BlockSpec((bm, 16), lambda i, j: (i, 0)),
                  pl.BlockSpec((16, bn), lambda i, j: (0, j))],
        out_specs=pl.BlockSpec((bm, bn), lambda i, j: (i, j)),
        out_shape=jax.ShapeDtypeStruct((N, N), jnp.float32),
    )(mu, mu_t)


def kernel(x, edge_index, edge_weight, W1, W2, W3):
    src = edge_index[0]
    dst = edge_index[1]
    pad = EPAD - E
    src3 = jnp.pad(src, (0, pad)).reshape(NW, CPW, CHUNK)
    dst3 = jnp.pad(dst, (0, pad)).reshape(NW, CPW, CHUNK)
    w3 = jnp.pad(edge_weight, (0, pad)).reshape(NW, CPW, CHUNK)

    sup1 = _tc_mm1(x, W1)
    p = _sc_spmm(sup1, src3, dst3, w3)
    w23 = jnp.concatenate([W2, W3], axis=1)
    sup23 = _tc_relu_mm(p[0], p[1], w23)
    q = _sc_spmm(sup23, src3, dst3, w3)
    z = _tc_add(q[0], q[1])
    mu = z[:, :16]
    logvar = z[:, 16:]
    adj = _tc_decoder(mu, mu.T)
    return adj, mu, logvar


# speedup vs baseline: 2.1457x; 1.7929x over previous
import functools

import jax
import jax.numpy as jnp
from jax import lax
from jax.experimental import pallas as pl
from jax.experimental.pallas import tpu as pltpu
from jax.experimental.pallas import tpu_sc as plsc

N = 10000
E = 320000
D_IN = 128
K = 32
KW = 128
NP = 2560
NC = 2
NS = 16
NW = NC * NS
CHUNK = 128
CPW = 80
EPW = CHUNK * CPW
EPAD = EPW * NW
RPT = NP // NS
NBUF = 2
PIECE = 16

_sc_mesh = plsc.VectorSubcoreMesh(core_axis_name="c", subcore_axis_name="s")


@functools.partial(
    pl.kernel,
    out_type=jax.ShapeDtypeStruct((NC, NP, KW), jnp.float32),
    mesh=_sc_mesh,
    scratch_types=[
        pltpu.VMEM((PIECE, CHUNK), jnp.int32),
        pltpu.VMEM((PIECE, CHUNK), jnp.int32),
        pltpu.VMEM((PIECE, CHUNK), jnp.int32),
        pltpu.VMEM((PIECE, CHUNK), jnp.int32),
        pltpu.VMEM((PIECE, CHUNK), jnp.float32),
        [pltpu.VMEM((CHUNK, KW), jnp.float32) for _ in range(NBUF)],
        [pltpu.VMEM((CHUNK, KW), jnp.float32) for _ in range(NBUF)],
        pltpu.VMEM((16, KW), jnp.float32),
        pltpu.VMEM_SHARED((NP, KW), jnp.float32),
        pltpu.VMEM_SHARED((NP, KW), jnp.float32),
        [pltpu.SemaphoreType.DMA for _ in range(NBUF)],
        [pltpu.SemaphoreType.DMA for _ in range(NBUF)],
    ],
)
def _sc_spmm(sup_hbm, srow_hbm, soff_hbm, drow_hbm, doff_hbm, w_hbm, out_hbm,
             srow_v, soff_v, drow_v, doff_v, w_v, gbuf, sbuf, zblk_v,
             table, accum, gsem, ssem):
    c = lax.axis_index("c")
    s = lax.axis_index("s")
    wid = c * NS + s

    toff = pl.multiple_of(s * RPT, 8)

    pltpu.sync_copy(sup_hbm.at[pl.ds(toff, RPT)], table.at[pl.ds(toff, RPT)])

    zv = jnp.zeros((16,), jnp.float32)
    for r in range(16):
        for q in range(KW // 16):
            zblk_v[r, pl.ds(q * 16, 16)] = zv

    def zero_body(i, carry):
        pltpu.sync_copy(
            zblk_v, accum.at[pl.ds(pl.multiple_of(toff + i * 16, 8), 16)])
        return carry

    lax.fori_loop(0, RPT // 16, zero_body, 0)
    plsc.subcore_barrier()

    def _gather(j, b):
        pltpu.async_copy(table.at[srow_v.at[j]], gbuf[b], gsem[b])

    def _gather_wait(j, b):
        pltpu.make_async_copy(table.at[srow_v.at[j]], gbuf[b],
                              gsem[b]).wait()

    def _scatter(j, b):
        pltpu.async_copy(sbuf[b], accum.at[drow_v.at[j]], ssem[b], add=True)

    def _scatter_wait(j, b):
        pltpu.make_async_copy(sbuf[b], accum.at[drow_v.at[j]],
                              ssem[b]).wait()

    def _scale(j, g, sb):
        def scale_body(gq, carry2):
            base = gq * 16
            wv = w_v[j, pl.ds(base, 16)]
            rv = soff_v[j, pl.ds(base, 16)]
            dv = doff_v[j, pl.ds(base, 16)]
            for l in range(16):
                row = base + l
                w = wv[l]
                so = rv[l]
                do = dv[l]
                o1 = (do + 32) % 128
                o2 = (do + 64) % 128
                o3 = (do + 96) % 128
                sb[row, pl.ds(do, 16)] = g[row, pl.ds(so, 16)] * w
                sb[row, pl.ds(do + 16, 16)] = g[row, pl.ds(so + 16, 16)] * w
                sb[row, pl.ds(o1, 16)] = zv
                sb[row, pl.ds(o1 + 16, 16)] = zv
                sb[row, pl.ds(o2, 16)] = zv
                sb[row, pl.ds(o2 + 16, 16)] = zv
                sb[row, pl.ds(o3, 16)] = zv
                sb[row, pl.ds(o3 + 16, 16)] = zv
            return carry2

        lax.fori_loop(0, CHUNK // 16, scale_body, 0)

    for h in range(CPW // PIECE):
        hp = h * PIECE
        pltpu.sync_copy(srow_hbm.at[wid, pl.ds(hp, PIECE)], srow_v)
        pltpu.sync_copy(soff_hbm.at[wid, pl.ds(hp, PIECE)], soff_v)
        pltpu.sync_copy(drow_hbm.at[wid, pl.ds(hp, PIECE)], drow_v)
        pltpu.sync_copy(doff_hbm.at[wid, pl.ds(hp, PIECE)], doff_v)
        pltpu.sync_copy(w_hbm.at[wid, pl.ds(hp, PIECE)], w_v)

        for b in range(NBUF):
            _gather(b, b)

        def chunk_body(jj, carry):
            for b in range(NBUF):
                j = jj * NBUF + b
                _gather_wait(j, b)

                @pl.when(jj > 0)
                def _():
                    _scatter_wait(j - NBUF, b)

                _scale(j, gbuf[b], sbuf[b])
                _scatter(j, b)

                @pl.when(jj < PIECE // NBUF - 1)
                def _():
                    _gather(j + NBUF, b)

            return carry

        lax.fori_loop(0, PIECE // NBUF, chunk_body, 0)
        for b in range(NBUF):
            _scatter_wait(PIECE - NBUF + b, b)
    plsc.subcore_barrier()

    pltpu.sync_copy(accum.at[pl.ds(toff, RPT)],
                    out_hbm.at[c, pl.ds(toff, RPT)])


def _mm1_body(x_ref, w_ref, o_ref):
    o_ref[...] = jnp.dot(x_ref[...], w_ref[...],
                         preferred_element_type=jnp.float32)


def _tc_mm1(x, w1):
    bm = 2000
    return pl.pallas_call(
        _mm1_body,
        grid=(N // bm,),
        in_specs=[pl.BlockSpec((bm, D_IN), lambda i: (i, 0)),
                  pl.BlockSpec((D_IN, K), lambda i: (0, 0))],
        out_specs=pl.BlockSpec((bm, K), lambda i: (i, 0)),
        out_shape=jax.ShapeDtypeStruct((N, K), jnp.float32),
    )(x, w1)


def _cmb_body(p0_ref, p1_ref, w_ref, o_ref):
    h = jnp.maximum(p0_ref[...] + p1_ref[...], 0.0)
    o_ref[...] = jnp.dot(h, w_ref[...], preferred_element_type=jnp.float32)


def _tc_relu_mm(p0, p1, w23):
    bm = 2000
    return pl.pallas_call(
        _cmb_body,
        grid=(N // bm,),
        in_specs=[pl.BlockSpec((bm, K), lambda i: (i, 0)),
                  pl.BlockSpec((bm, K), lambda i: (i, 0)),
                  pl.BlockSpec((K, K), lambda i: (0, 0))],
        out_specs=pl.BlockSpec((bm, K), lambda i: (i, 0)),
        out_shape=jax.ShapeDtypeStruct((N, K), jnp.float32),
    )(p0, p1, w23)


def _add_body(a_ref, b_ref, o_ref):
    o_ref[...] = a_ref[...] + b_ref[...]


def _tc_add(a, b):
    bm = 2000
    return pl.pallas_call(
        _add_body,
        grid=(N // bm,),
        in_specs=[pl.BlockSpec((bm, K), lambda i: (i, 0)),
                  pl.BlockSpec((bm, K), lambda i: (i, 0))],
        out_specs=pl.BlockSpec((bm, K), lambda i: (i, 0)),
        out_shape=jax.ShapeDtypeStruct((N, K), jnp.float32),
    )(a, b)


def _dec_body(a_ref, bt_ref, o_ref):
    o_ref[...] = jax.nn.sigmoid(
        jnp.dot(a_ref[...], bt_ref[...], preferred_element_type=jnp.float32))


def _tc_decoder(mu, mu_t):
    bm = 1024
    bn = 1024
    return pl.pallas_call(
        _dec_body,
        grid=(pl.cdiv(N, bm), pl.cdiv(N, bn)),
        in_specs=[pl.BlockSpec((bm, 16), lambda i, j: (i, 0)),
                  pl.BlockSpec((16, bn), lambda i, j: (0, j))],
        out_specs=pl.BlockSpec((bm, bn), lambda i, j: (i, j)),
        out_shape=jax.ShapeDtypeStruct((N, N), jnp.float32),
    )(mu, mu_t)


def _pack(sup):
    return jnp.pad(sup.reshape(N // 4, KW), ((0, NP - N // 4), (0, 0)))


def _unpack(part):
    return part[: N // 4, :].reshape(N, K)


def kernel(x, edge_index, edge_weight, W1, W2, W3):
    src = edge_index[0]
    dst = edge_index[1]
    pad = EPAD - E
    srow3 = jnp.pad(src >> 2, (0, pad)).reshape(NW, CPW, CHUNK)
    soff3 = jnp.pad((src & 3) * K, (0, pad)).reshape(NW, CPW, CHUNK)
    drow3 = jnp.pad(dst >> 2, (0, pad)).reshape(NW, CPW, CHUNK)
    doff3 = jnp.pad((dst & 3) * K, (0, pad)).reshape(NW, CPW, CHUNK)
    w3 = jnp.pad(edge_weight, (0, pad)).reshape(NW, CPW, CHUNK)

    sup1 = _tc_mm1(x, W1)
    p = _sc_spmm(_pack(sup1), srow3, soff3, drow3, doff3, w3)
    w23 = jnp.concatenate([W2, W3], axis=1)
    sup23 = _tc_relu_mm(_unpack(p[0]), _unpack(p[1]), w23)
    q = _sc_spmm(_pack(sup23), srow3, soff3, drow3, doff3, w3)
    z = _tc_add(_unpack(q[0]), _unpack(q[1]))
    mu = z[:, :16]
    logvar = z[:, 16:]
    adj = _tc_decoder(mu, mu.T)
    return adj, mu, logvar


# speedup vs baseline: 2.2281x; 1.0384x over previous
import functools

import jax
import jax.numpy as jnp
from jax import lax
from jax.experimental import pallas as pl
from jax.experimental.pallas import tpu as pltpu
from jax.experimental.pallas import tpu_sc as plsc

N = 10000
E = 320000
D_IN = 128
K = 32
KW = 128
NP = 2560
NC = 2
NS = 16
NW = NC * NS
CHUNK = 128
CPW = 80
EPW = CHUNK * CPW
EPAD = EPW * NW
RPT = NP // NS
NBUF = 2
PIECE = 16

_sc_mesh = plsc.VectorSubcoreMesh(core_axis_name="c", subcore_axis_name="s")


@functools.partial(
    pl.kernel,
    out_type=jax.ShapeDtypeStruct((NC, NP, KW), jnp.float32),
    mesh=_sc_mesh,
    scratch_types=[
        pltpu.VMEM((PIECE, CHUNK), jnp.int32),
        pltpu.VMEM((PIECE, CHUNK), jnp.int32),
        pltpu.VMEM((PIECE, CHUNK), jnp.int32),
        pltpu.VMEM((PIECE, CHUNK), jnp.int32),
        pltpu.VMEM((PIECE, CHUNK), jnp.float32),
        [pltpu.VMEM((CHUNK, KW), jnp.float32) for _ in range(NBUF)],
        [pltpu.VMEM((CHUNK, KW), jnp.float32) for _ in range(NBUF)],
        pltpu.VMEM((16, KW), jnp.float32),
        pltpu.VMEM_SHARED((NP, KW), jnp.float32),
        pltpu.VMEM_SHARED((NP, KW), jnp.float32),
        [pltpu.SemaphoreType.DMA for _ in range(NBUF)],
        [pltpu.SemaphoreType.DMA for _ in range(NBUF)],
    ],
)
def _sc_spmm(sup_hbm, srow_hbm, soff_hbm, drow_hbm, doff_hbm, w_hbm, out_hbm,
             srow_v, soff_v, drow_v, doff_v, w_v, gbuf, sbuf, zblk_v,
             table, accum, gsem, ssem):
    c = lax.axis_index("c")
    s = lax.axis_index("s")
    wid = c * NS + s

    toff = pl.multiple_of(s * RPT, 8)

    pltpu.sync_copy(sup_hbm.at[pl.ds(toff, RPT)], table.at[pl.ds(toff, RPT)])

    zv = jnp.zeros((16,), jnp.float32)
    for r in range(16):
        for q in range(KW // 16):
            zblk_v[r, pl.ds(q * 16, 16)] = zv

    def zero_body(i, carry):
        pltpu.sync_copy(
            zblk_v, accum.at[pl.ds(pl.multiple_of(toff + i * 16, 8), 16)])
        return carry

    lax.fori_loop(0, RPT // 16, zero_body, 0)
    plsc.subcore_barrier()

    def _gather(j, b):
        pltpu.async_copy(table.at[srow_v.at[j]], gbuf[b], gsem[b])

    def _gather_wait(j, b):
        pltpu.make_async_copy(table.at[srow_v.at[j]], gbuf[b],
                              gsem[b]).wait()

    def _scatter(j, b):
        pltpu.async_copy(sbuf[b], accum.at[drow_v.at[j]], ssem[b], add=True)

    def _scatter_wait(j, b):
        pltpu.make_async_copy(sbuf[b], accum.at[drow_v.at[j]],
                              ssem[b]).wait()

    def _scale_full(j, g, sb):
        def scale_body(gq, carry2):
            base = gq * 16
            wv = w_v[j, pl.ds(base, 16)]
            rv = soff_v[j, pl.ds(base, 16)]
            dv = doff_v[j, pl.ds(base, 16)]
            for l in range(16):
                row = base + l
                w = wv[l]
                so = rv[l]
                do = dv[l]
                o1 = (do + 32) % 128
                o2 = (do + 64) % 128
                o3 = (do + 96) % 128
                sb[row, pl.ds(do, 16)] = g[row, pl.ds(so, 16)] * w
                sb[row, pl.ds(do + 16, 16)] = g[row, pl.ds(so + 16, 16)] * w
                sb[row, pl.ds(o1, 16)] = zv
                sb[row, pl.ds(o1 + 16, 16)] = zv
                sb[row, pl.ds(o2, 16)] = zv
                sb[row, pl.ds(o2 + 16, 16)] = zv
                sb[row, pl.ds(o3, 16)] = zv
                sb[row, pl.ds(o3 + 16, 16)] = zv
            return carry2

        lax.fori_loop(0, CHUNK // 16, scale_body, 0)

    def _scale_prev(j, g, sb):
        def scale_body(gq, carry2):
            base = gq * 16
            wv = w_v[j, pl.ds(base, 16)]
            rv = soff_v[j, pl.ds(base, 16)]
            dv = doff_v[j, pl.ds(base, 16)]
            pv = doff_v[j - NBUF, pl.ds(base, 16)]
            for l in range(16):
                row = base + l
                w = wv[l]
                so = rv[l]
                do = dv[l]
                po = pv[l]
                sb[row, pl.ds(po, 16)] = zv
                sb[row, pl.ds(po + 16, 16)] = zv
                sb[row, pl.ds(do, 16)] = g[row, pl.ds(so, 16)] * w
                sb[row, pl.ds(do + 16, 16)] = g[row, pl.ds(so + 16, 16)] * w
            return carry2

        lax.fori_loop(0, CHUNK // 16, scale_body, 0)

    for h in range(CPW // PIECE):
        hp = h * PIECE
        pltpu.sync_copy(srow_hbm.at[wid, pl.ds(hp, PIECE)], srow_v)
        pltpu.sync_copy(soff_hbm.at[wid, pl.ds(hp, PIECE)], soff_v)
        pltpu.sync_copy(drow_hbm.at[wid, pl.ds(hp, PIECE)], drow_v)
        pltpu.sync_copy(doff_hbm.at[wid, pl.ds(hp, PIECE)], doff_v)
        pltpu.sync_copy(w_hbm.at[wid, pl.ds(hp, PIECE)], w_v)

        for b in range(NBUF):
            _gather(b, b)

        def chunk_body(jj, carry):
            for b in range(NBUF):
                j = jj * NBUF + b
                _gather_wait(j, b)

                @pl.when(jj > 0)
                def _():
                    _scatter_wait(j - NBUF, b)
                    _scale_prev(j, gbuf[b], sbuf[b])

                @pl.when(jj == 0)
                def _():
                    _scale_full(j, gbuf[b], sbuf[b])

                _scatter(j, b)

                @pl.when(jj < PIECE // NBUF - 1)
                def _():
                    _gather(j + NBUF, b)

            return carry

        lax.fori_loop(0, PIECE // NBUF, chunk_body, 0)
        for b in range(NBUF):
            _scatter_wait(PIECE - NBUF + b, b)
    plsc.subcore_barrier()

    pltpu.sync_copy(accum.at[pl.ds(toff, RPT)],
                    out_hbm.at[c, pl.ds(toff, RPT)])


def _mm1_body(x_ref, w_ref, o_ref):
    o_ref[...] = jnp.dot(x_ref[...], w_ref[...],
                         preferred_element_type=jnp.float32)


def _tc_mm1(x, w1):
    bm = 2000
    return pl.pallas_call(
        _mm1_body,
        grid=(N // bm,),
        in_specs=[pl.BlockSpec((bm, D_IN), lambda i: (i, 0)),
                  pl.BlockSpec((D_IN, K), lambda i: (0, 0))],
        out_specs=pl.BlockSpec((bm, K), lambda i: (i, 0)),
        out_shape=jax.ShapeDtypeStruct((N, K), jnp.float32),
    )(x, w1)


def _cmb_body(p0_ref, p1_ref, w_ref, o_ref):
    h = jnp.maximum(p0_ref[...] + p1_ref[...], 0.0)
    o_ref[...] = jnp.dot(h, w_ref[...], preferred_element_type=jnp.float32)


def _tc_relu_mm(p0, p1, w23):
    bm = 2000
    return pl.pallas_call(
        _cmb_body,
        grid=(N // bm,),
        in_specs=[pl.BlockSpec((bm, K), lambda i: (i, 0)),
                  pl.BlockSpec((bm, K), lambda i: (i, 0)),
                  pl.BlockSpec((K, K), lambda i: (0, 0))],
        out_specs=pl.BlockSpec((bm, K), lambda i: (i, 0)),
        out_shape=jax.ShapeDtypeStruct((N, K), jnp.float32),
    )(p0, p1, w23)


def _add_body(a_ref, b_ref, o_ref):
    o_ref[...] = a_ref[...] + b_ref[...]


def _tc_add(a, b):
    bm = 2000
    return pl.pallas_call(
        _add_body,
        grid=(N // bm,),
        in_specs=[pl.BlockSpec((bm, K), lambda i: (i, 0)),
                  pl.BlockSpec((bm, K), lambda i: (i, 0))],
        out_specs=pl.BlockSpec((bm, K), lambda i: (i, 0)),
        out_shape=jax.ShapeDtypeStruct((N, K), jnp.float32),
    )(a, b)


def _dec_body(a_ref, bt_ref, o_ref):
    o_ref[...] = jax.nn.sigmoid(
        jnp.dot(a_ref[...], bt_ref[...], preferred_element_type=jnp.float32))


def _tc_decoder(mu, mu_t):
    bm = 1024
    bn = 1024
    return pl.pallas_call(
        _dec_body,
        grid=(pl.cdiv(N, bm), pl.cdiv(N, bn)),
        in_specs=[pl.BlockSpec((bm, 16), lambda i, j: (i, 0)),
                  pl.BlockSpec((16, bn), lambda i, j: (0, j))],
        out_specs=pl.BlockSpec((bm, bn), lambda i, j: (i, j)),
        out_shape=jax.ShapeDtypeStruct((N, N), jnp.float32),
    )(mu, mu_t)


def _pack(sup):
    return jnp.pad(sup.reshape(N // 4, KW), ((0, NP - N // 4), (0, 0)))


def _unpack(part):
    return part[: N // 4, :].reshape(N, K)


def kernel(x, edge_index, edge_weight, W1, W2, W3):
    src = edge_index[0]
    dst = edge_index[1]
    pad = EPAD - E
    srow3 = jnp.pad(src >> 2, (0, pad)).reshape(NW, CPW, CHUNK)
    soff3 = jnp.pad((src & 3) * K, (0, pad)).reshape(NW, CPW, CHUNK)
    drow3 = jnp.pad(dst >> 2, (0, pad)).reshape(NW, CPW, CHUNK)
    doff3 = jnp.pad((dst & 3) * K, (0, pad)).reshape(NW, CPW, CHUNK)
    w3 = jnp.pad(edge_weight, (0, pad)).reshape(NW, CPW, CHUNK)

    sup1 = _tc_mm1(x, W1)
    p = _sc_spmm(_pack(sup1), srow3, soff3, drow3, doff3, w3)
    w23 = jnp.concatenate([W2, W3], axis=1)
    sup23 = _tc_relu_mm(_unpack(p[0]), _unpack(p[1]), w23)
    q = _sc_spmm(_pack(sup23), srow3, soff3, drow3, doff3, w3)
    z = _tc_add(_unpack(q[0]), _unpack(q[1]))
    mu = z[:, :16]
    logvar = z[:, 16:]
    adj = _tc_decoder(mu, mu.T)
    return adj, mu, logvar


# speedup vs baseline: 2.3236x; 1.0428x over previous
import functools

import jax
import jax.numpy as jnp
from jax import lax
from jax.experimental import pallas as pl
from jax.experimental.pallas import tpu as pltpu
from jax.experimental.pallas import tpu_sc as plsc

N = 10000
E = 320000
D_IN = 128
K = 32
KW = 128
NP = 2560
NC = 2
NS = 16
NW = NC * NS
CHUNK = 128
CPW = 80
EPW = CHUNK * CPW
EPAD = EPW * NW
RPT = NP // NS
NBUF = 2
PIECE = 16

_sc_mesh = plsc.VectorSubcoreMesh(core_axis_name="c", subcore_axis_name="s")


@functools.partial(
    pl.kernel,
    out_type=jax.ShapeDtypeStruct((NC, NP, KW), jnp.float32),
    mesh=_sc_mesh,
    scratch_types=[
        pltpu.VMEM((PIECE, CHUNK), jnp.int32),
        pltpu.VMEM((PIECE, CHUNK), jnp.int32),
        pltpu.VMEM((PIECE, CHUNK), jnp.int32),
        pltpu.VMEM((PIECE, CHUNK), jnp.int32),
        pltpu.VMEM((PIECE, CHUNK), jnp.float32),
        [pltpu.VMEM((CHUNK, KW), jnp.float32) for _ in range(NBUF)],
        [pltpu.VMEM((CHUNK, KW), jnp.float32) for _ in range(NBUF)],
        pltpu.VMEM((16, KW), jnp.float32),
        pltpu.VMEM_SHARED((NP, KW), jnp.float32),
        pltpu.VMEM_SHARED((NP, KW), jnp.float32),
        [pltpu.SemaphoreType.DMA for _ in range(NBUF)],
        [pltpu.SemaphoreType.DMA for _ in range(NBUF)],
    ],
)
def _sc_spmm(sup_hbm, srow_hbm, soff_hbm, drow_hbm, doff_hbm, w_hbm, out_hbm,
             srow_v, soff_v, drow_v, doff_v, w_v, gbuf, sbuf, zblk_v,
             table, accum, gsem, ssem):
    c = lax.axis_index("c")
    s = lax.axis_index("s")
    wid = c * NS + s

    toff = pl.multiple_of(s * RPT, 8)

    pltpu.sync_copy(sup_hbm.at[pl.ds(toff, RPT)], table.at[pl.ds(toff, RPT)])

    zv = jnp.zeros((16,), jnp.float32)
    for r in range(16):
        for q in range(KW // 16):
            zblk_v[r, pl.ds(q * 16, 16)] = zv

    def zero_body(i, carry):
        pltpu.sync_copy(
            zblk_v, accum.at[pl.ds(pl.multiple_of(toff + i * 16, 8), 16)])
        return carry

    lax.fori_loop(0, RPT // 16, zero_body, 0)
    plsc.subcore_barrier()

    def _gather(j, b):
        pltpu.async_copy(table.at[srow_v.at[j]], gbuf[b], gsem[b])

    def _gather_wait(j, b):
        pltpu.make_async_copy(table.at[srow_v.at[j]], gbuf[b],
                              gsem[b]).wait()

    def _scatter(j, b):
        pltpu.async_copy(sbuf[b], accum.at[drow_v.at[j]], ssem[b], add=True)

    def _scatter_wait(j, b):
        pltpu.make_async_copy(sbuf[b], accum.at[drow_v.at[j]],
                              ssem[b]).wait()

    def _scale_full(j, g, sb):
        def scale_body(gq, carry2):
            base = gq * 16
            wv = w_v[j, pl.ds(base, 16)]
            rv = soff_v[j, pl.ds(base, 16)]
            dv = doff_v[j, pl.ds(base, 16)]
            for l in range(16):
                row = base + l
                w = wv[l]
                so = rv[l]
                do = dv[l]
                o1 = (do + 32) % 128
                o2 = (do + 64) % 128
                o3 = (do + 96) % 128
                sb[row, pl.ds(do, 16)] = g[row, pl.ds(so, 16)] * w
                sb[row, pl.ds(do + 16, 16)] = g[row, pl.ds(so + 16, 16)] * w
                sb[row, pl.ds(o1, 16)] = zv
                sb[row, pl.ds(o1 + 16, 16)] = zv
                sb[row, pl.ds(o2, 16)] = zv
                sb[row, pl.ds(o2 + 16, 16)] = zv
                sb[row, pl.ds(o3, 16)] = zv
                sb[row, pl.ds(o3 + 16, 16)] = zv
            return carry2

        lax.fori_loop(0, CHUNK // 16, scale_body, 0)

    def _scale_prev(j, g, sb):
        def scale_body(gq, carry2):
            base = gq * 16
            wv = w_v[j, pl.ds(base, 16)]
            rv = soff_v[j, pl.ds(base, 16)]
            dv = doff_v[j, pl.ds(base, 16)]
            pv = doff_v[j - NBUF, pl.ds(base, 16)]
            for l in range(16):
                row = base + l
                w = wv[l]
                so = rv[l]
                do = dv[l]
                po = pv[l]
                sb[row, pl.ds(po, 16)] = zv
                sb[row, pl.ds(po + 16, 16)] = zv
                sb[row, pl.ds(do, 16)] = g[row, pl.ds(so, 16)] * w
                sb[row, pl.ds(do + 16, 16)] = g[row, pl.ds(so + 16, 16)] * w
            return carry2

        lax.fori_loop(0, CHUNK // 16, scale_body, 0)

    for h in range(CPW // PIECE):
        hp = h * PIECE
        pltpu.sync_copy(srow_hbm.at[wid, pl.ds(hp, PIECE)], srow_v)
        pltpu.sync_copy(soff_hbm.at[wid, pl.ds(hp, PIECE)], soff_v)
        pltpu.sync_copy(drow_hbm.at[wid, pl.ds(hp, PIECE)], drow_v)
        pltpu.sync_copy(doff_hbm.at[wid, pl.ds(hp, PIECE)], doff_v)
        pltpu.sync_copy(w_hbm.at[wid, pl.ds(hp, PIECE)], w_v)

        for b in range(NBUF):
            _gather(b, b)

        def chunk_body(jj, carry):
            for b in range(NBUF):
                j = jj * NBUF + b
                _gather_wait(j, b)

                @pl.when(jj > 0)
                def _():
                    _scatter_wait(j - NBUF, b)
                    _scale_prev(j, gbuf[b], sbuf[b])

                @pl.when(jj == 0)
                def _():
                    _scale_full(j, gbuf[b], sbuf[b])

                _scatter(j, b)

                @pl.when(jj < PIECE // NBUF - 1)
                def _():
                    _gather(j + NBUF, b)

            return carry

        lax.fori_loop(0, PIECE // NBUF, chunk_body, 0)
        for b in range(NBUF):
            _scatter_wait(PIECE - NBUF + b, b)
    plsc.subcore_barrier()

    pltpu.sync_copy(accum.at[pl.ds(toff, RPT)],
                    out_hbm.at[c, pl.ds(toff, RPT)])


def _mm1_body(x_ref, w_ref, o_ref):
    o_ref[...] = jnp.dot(x_ref[...], w_ref[...],
                         preferred_element_type=jnp.float32)


def _tc_mm1(x, w1):
    bm = 2000
    return pl.pallas_call(
        _mm1_body,
        grid=(N // bm,),
        in_specs=[pl.BlockSpec((bm, D_IN), lambda i: (i, 0)),
                  pl.BlockSpec((D_IN, K), lambda i: (0, 0))],
        out_specs=pl.BlockSpec((bm, K), lambda i: (i, 0)),
        out_shape=jax.ShapeDtypeStruct((N, K), jnp.float32),
    )(x, w1)


def _cmb_body(p0_ref, p1_ref, w_ref, o_ref):
    h = jnp.maximum(p0_ref[...] + p1_ref[...], 0.0)
    o_ref[...] = jnp.dot(h, w_ref[...], preferred_element_type=jnp.float32)


def _tc_relu_mm(p0, p1, w23):
    bm = 2000
    return pl.pallas_call(
        _cmb_body,
        grid=(N // bm,),
        in_specs=[pl.BlockSpec((bm, K), lambda i: (i, 0)),
                  pl.BlockSpec((bm, K), lambda i: (i, 0)),
                  pl.BlockSpec((K, K), lambda i: (0, 0))],
        out_specs=pl.BlockSpec((bm, K), lambda i: (i, 0)),
        out_shape=jax.ShapeDtypeStruct((N, K), jnp.float32),
    )(p0, p1, w23)


def _add_body(a_ref, b_ref, o_ref):
    o_ref[...] = a_ref[...] + b_ref[...]


def _tc_add(a, b):
    bm = 2000
    return pl.pallas_call(
        _add_body,
        grid=(N // bm,),
        in_specs=[pl.BlockSpec((bm, K), lambda i: (i, 0)),
                  pl.BlockSpec((bm, K), lambda i: (i, 0))],
        out_specs=pl.BlockSpec((bm, K), lambda i: (i, 0)),
        out_shape=jax.ShapeDtypeStruct((N, K), jnp.float32),
    )(a, b)


def _dec_body(a_ref, bt_ref, o_ref):
    o_ref[...] = jax.nn.sigmoid(
        jnp.dot(a_ref[...], bt_ref[...], preferred_element_type=jnp.float32))


def _tc_decoder(mu, mu_t):
    bm = 1024
    bn = 2048
    return pl.pallas_call(
        _dec_body,
        grid=(pl.cdiv(N, bm), pl.cdiv(N, bn)),
        in_specs=[pl.BlockSpec((bm, 16), lambda i, j: (i, 0)),
                  pl.BlockSpec((16, bn), lambda i, j: (0, j))],
        out_specs=pl.BlockSpec((bm, bn), lambda i, j: (i, j)),
        out_shape=jax.ShapeDtypeStruct((N, N), jnp.float32),
    )(mu, mu_t)


def _pack(sup):
    return jnp.pad(sup.reshape(N // 4, KW), ((0, NP - N // 4), (0, 0)))


def _unpack(part):
    return part[: N // 4, :].reshape(N, K)


def kernel(x, edge_index, edge_weight, W1, W2, W3):
    src = edge_index[0]
    dst = edge_index[1]
    pad = EPAD - E
    srow3 = jnp.pad(src >> 2, (0, pad)).reshape(NW, CPW, CHUNK)
    soff3 = jnp.pad((src & 3) * K, (0, pad)).reshape(NW, CPW, CHUNK)
    drow3 = jnp.pad(dst >> 2, (0, pad)).reshape(NW, CPW, CHUNK)
    doff3 = jnp.pad((dst & 3) * K, (0, pad)).reshape(NW, CPW, CHUNK)
    w3 = jnp.pad(edge_weight, (0, pad)).reshape(NW, CPW, CHUNK)

    sup1 = _tc_mm1(x, W1)
    p = _sc_spmm(_pack(sup1), srow3, soff3, drow3, doff3, w3)
    w23 = jnp.concatenate([W2, W3], axis=1)
    sup23 = _tc_relu_mm(_unpack(p[0]), _unpack(p[1]), w23)
    q = _sc_spmm(_pack(sup23), srow3, soff3, drow3, doff3, w3)
    z = _tc_add(_unpack(q[0]), _unpack(q[1]))
    mu = z[:, :16]
    logvar = z[:, 16:]
    adj = _tc_decoder(mu, mu.T)
    return adj, mu, logvar


# speedup vs baseline: 2.3463x; 1.0098x over previous
import functools

import jax
import jax.numpy as jnp
from jax import lax
from jax.experimental import pallas as pl
from jax.experimental.pallas import tpu as pltpu
from jax.experimental.pallas import tpu_sc as plsc

N = 10000
E = 320000
D_IN = 128
K = 32
KW = 128
NP = 2560
NC = 2
NS = 16
NW = NC * NS
CHUNK = 128
CPW = 80
EPW = CHUNK * CPW
EPAD = EPW * NW
RPT = NP // NS
NBUF = 2
PIECE = 16

_sc_mesh = plsc.VectorSubcoreMesh(core_axis_name="c", subcore_axis_name="s")


@functools.partial(
    pl.kernel,
    out_type=jax.ShapeDtypeStruct((NC, NP, KW), jnp.float32),
    mesh=_sc_mesh,
    scratch_types=[
        pltpu.VMEM((PIECE, CHUNK), jnp.int32),
        pltpu.VMEM((PIECE, CHUNK), jnp.int32),
        pltpu.VMEM((PIECE, CHUNK), jnp.int32),
        pltpu.VMEM((PIECE, CHUNK), jnp.int32),
        pltpu.VMEM((PIECE, CHUNK), jnp.float32),
        [pltpu.VMEM((CHUNK, KW), jnp.float32) for _ in range(NBUF)],
        [pltpu.VMEM((CHUNK, KW), jnp.float32) for _ in range(NBUF)],
        pltpu.VMEM((16, KW), jnp.float32),
        pltpu.VMEM_SHARED((NP, KW), jnp.float32),
        pltpu.VMEM_SHARED((NP, KW), jnp.float32),
        [pltpu.SemaphoreType.DMA for _ in range(NBUF)],
        [pltpu.SemaphoreType.DMA for _ in range(NBUF)],
    ],
)
def _sc_spmm(sup_hbm, srow_hbm, soff_hbm, drow_hbm, doff_hbm, w_hbm, out_hbm,
             srow_v, soff_v, drow_v, doff_v, w_v, gbuf, sbuf, zblk_v,
             table, accum, gsem, ssem):
    c = lax.axis_index("c")
    s = lax.axis_index("s")
    wid = c * NS + s

    toff = pl.multiple_of(s * RPT, 8)

    pltpu.sync_copy(sup_hbm.at[pl.ds(toff, RPT)], table.at[pl.ds(toff, RPT)])

    zv = jnp.zeros((16,), jnp.float32)
    for r in range(16):
        for q in range(KW // 16):
            zblk_v[r, pl.ds(q * 16, 16)] = zv

    def zero_body(i, carry):
        pltpu.sync_copy(
            zblk_v, accum.at[pl.ds(pl.multiple_of(toff + i * 16, 8), 16)])
        return carry

    lax.fori_loop(0, RPT // 16, zero_body, 0)
    plsc.subcore_barrier()

    def _gather(j, b):
        pltpu.async_copy(table.at[srow_v.at[j]], gbuf[b], gsem[b])

    def _gather_wait(j, b):
        pltpu.make_async_copy(table.at[srow_v.at[j]], gbuf[b],
                              gsem[b]).wait()

    def _scatter(j, b):
        pltpu.async_copy(sbuf[b], accum.at[drow_v.at[j]], ssem[b], add=True)

    def _scatter_wait(j, b):
        pltpu.make_async_copy(sbuf[b], accum.at[drow_v.at[j]],
                              ssem[b]).wait()

    def _scale_full(j, g, sb):
        def scale_body(gq, carry2):
            base = gq * 16
            wv = w_v[j, pl.ds(base, 16)]
            rv = soff_v[j, pl.ds(base, 16)]
            dv = doff_v[j, pl.ds(base, 16)]
            for l in range(16):
                row = base + l
                w = wv[l]
                so = rv[l]
                do = dv[l]
                o1 = (do + 32) % 128
                o2 = (do + 64) % 128
                o3 = (do + 96) % 128
                sb[row, pl.ds(do, 16)] = g[row, pl.ds(so, 16)] * w
                sb[row, pl.ds(do + 16, 16)] = g[row, pl.ds(so + 16, 16)] * w
                sb[row, pl.ds(o1, 16)] = zv
                sb[row, pl.ds(o1 + 16, 16)] = zv
                sb[row, pl.ds(o2, 16)] = zv
                sb[row, pl.ds(o2 + 16, 16)] = zv
                sb[row, pl.ds(o3, 16)] = zv
                sb[row, pl.ds(o3 + 16, 16)] = zv
            return carry2

        lax.fori_loop(0, CHUNK // 16, scale_body, 0)

    def _scale_prev(j, g, sb):
        def scale_body(gq, carry2):
            base = gq * 16
            wv = w_v[j, pl.ds(base, 16)]
            rv = soff_v[j, pl.ds(base, 16)]
            dv = doff_v[j, pl.ds(base, 16)]
            pv = doff_v[j - NBUF, pl.ds(base, 16)]
            for l in range(16):
                row = base + l
                w = wv[l]
                so = rv[l]
                do = dv[l]
                po = pv[l]
                sb[row, pl.ds(po, 16)] = zv
                sb[row, pl.ds(po + 16, 16)] = zv
                sb[row, pl.ds(do, 16)] = g[row, pl.ds(so, 16)] * w
                sb[row, pl.ds(do + 16, 16)] = g[row, pl.ds(so + 16, 16)] * w
            return carry2

        lax.fori_loop(0, CHUNK // 16, scale_body, 0)

    for h in range(CPW // PIECE):
        hp = h * PIECE
        pltpu.sync_copy(srow_hbm.at[wid, pl.ds(hp, PIECE)], srow_v)
        pltpu.sync_copy(soff_hbm.at[wid, pl.ds(hp, PIECE)], soff_v)
        pltpu.sync_copy(drow_hbm.at[wid, pl.ds(hp, PIECE)], drow_v)
        pltpu.sync_copy(doff_hbm.at[wid, pl.ds(hp, PIECE)], doff_v)
        pltpu.sync_copy(w_hbm.at[wid, pl.ds(hp, PIECE)], w_v)

        for b in range(NBUF):
            _gather(b, b)

        def chunk_body(jj, carry):
            for b in range(NBUF):
                j = jj * NBUF + b
                _gather_wait(j, b)

                @pl.when(jj > 0)
                def _():
                    _scatter_wait(j - NBUF, b)
                    _scale_prev(j, gbuf[b], sbuf[b])

                @pl.when(jj == 0)
                def _():
                    _scale_full(j, gbuf[b], sbuf[b])

                _scatter(j, b)

                @pl.when(jj < PIECE // NBUF - 1)
                def _():
                    _gather(j + NBUF, b)

            return carry

        lax.fori_loop(0, PIECE // NBUF, chunk_body, 0)
        for b in range(NBUF):
            _scatter_wait(PIECE - NBUF + b, b)
    plsc.subcore_barrier()

    pltpu.sync_copy(accum.at[pl.ds(toff, RPT)],
                    out_hbm.at[c, pl.ds(toff, RPT)])


def _mm1_body(x_ref, w_ref, o_ref):
    o_ref[...] = jnp.dot(x_ref[...], w_ref[...],
                         preferred_element_type=jnp.float32)


def _tc_mm1(x, w1):
    bm = 2000
    return pl.pallas_call(
        _mm1_body,
        grid=(N // bm,),
        in_specs=[pl.BlockSpec((bm, D_IN), lambda i: (i, 0)),
                  pl.BlockSpec((D_IN, K), lambda i: (0, 0))],
        out_specs=pl.BlockSpec((bm, K), lambda i: (i, 0)),
        out_shape=jax.ShapeDtypeStruct((N, K), jnp.float32),
    )(x, w1)


def _cmb_body(p0_ref, p1_ref, w_ref, o_ref):
    h = jnp.maximum(p0_ref[...] + p1_ref[...], 0.0)
    o_ref[...] = jnp.dot(h, w_ref[...], preferred_element_type=jnp.float32)


def _tc_relu_mm(p0, p1, w23):
    bm = 2000
    return pl.pallas_call(
        _cmb_body,
        grid=(N // bm,),
        in_specs=[pl.BlockSpec((bm, K), lambda i: (i, 0)),
                  pl.BlockSpec((bm, K), lambda i: (i, 0)),
                  pl.BlockSpec((K, K), lambda i: (0, 0))],
        out_specs=pl.BlockSpec((bm, K), lambda i: (i, 0)),
        out_shape=jax.ShapeDtypeStruct((N, K), jnp.float32),
    )(p0, p1, w23)


def _add_body(a_ref, b_ref, o_ref):
    o_ref[...] = a_ref[...] + b_ref[...]


def _tc_add(a, b):
    bm = 2000
    return pl.pallas_call(
        _add_body,
        grid=(N // bm,),
        in_specs=[pl.BlockSpec((bm, K), lambda i: (i, 0)),
                  pl.BlockSpec((bm, K), lambda i: (i, 0))],
        out_specs=pl.BlockSpec((bm, K), lambda i: (i, 0)),
        out_shape=jax.ShapeDtypeStruct((N, K), jnp.float32),
    )(a, b)


def _dec_body(a_ref, bt_ref, o_ref):
    o_ref[...] = jax.nn.sigmoid(
        jnp.dot(a_ref[...], bt_ref[...], preferred_element_type=jnp.float32))


def _tc_decoder(mu, mu_t):
    bm = 2048
    bn = 2048
    return pl.pallas_call(
        _dec_body,
        grid=(pl.cdiv(N, bm), pl.cdiv(N, bn)),
        in_specs=[pl.BlockSpec((bm, 16), lambda i, j: (i, 0)),
                  pl.BlockSpec((16, bn), lambda i, j: (0, j))],
        out_specs=pl.BlockSpec((bm, bn), lambda i, j: (i, j)),
        out_shape=jax.ShapeDtypeStruct((N, N), jnp.float32),
    )(mu, mu_t)


def _pack(sup):
    return jnp.pad(sup.reshape(N // 4, KW), ((0, NP - N // 4), (0, 0)))


def _unpack(part):
    return part[: N // 4, :].reshape(N, K)


def kernel(x, edge_index, edge_weight, W1, W2, W3):
    src = edge_index[0]
    dst = edge_index[1]
    pad = EPAD - E
    srow3 = jnp.pad(src >> 2, (0, pad)).reshape(NW, CPW, CHUNK)
    soff3 = jnp.pad((src & 3) * K, (0, pad)).reshape(NW, CPW, CHUNK)
    drow3 = jnp.pad(dst >> 2, (0, pad)).reshape(NW, CPW, CHUNK)
    doff3 = jnp.pad((dst & 3) * K, (0, pad)).reshape(NW, CPW, CHUNK)
    w3 = jnp.pad(edge_weight, (0, pad)).reshape(NW, CPW, CHUNK)

    sup1 = _tc_mm1(x, W1)
    p = _sc_spmm(_pack(sup1), srow3, soff3, drow3, doff3, w3)
    w23 = jnp.concatenate([W2, W3], axis=1)
    sup23 = _tc_relu_mm(_unpack(p[0]), _unpack(p[1]), w23)
    q = _sc_spmm(_pack(sup23), srow3, soff3, drow3, doff3, w3)
    z = _tc_add(_unpack(q[0]), _unpack(q[1]))
    mu = z[:, :16]
    logvar = z[:, 16:]
    adj = _tc_decoder(mu, mu.T)
    return adj, mu, logvar
